# baseline (speedup 1.0000x reference)
import functools
import jax
import jax.numpy as jnp
from jax.experimental import pallas as pl
from jax.experimental.pallas import tpu as pltpu

_M = 256


def _dense_body(x_ref, wg_ref, w1_ref, w2_ref, out_ref, w_scr):
    e = pl.program_id(1)
    n_e = pl.num_programs(1)
    x = x_ref[...]

    @pl.when(e == 0)
    def _():
        logits = jax.lax.dot_general(
            x, wg_ref[...], (((1,), (1,)), ((), ())),
            preferred_element_type=jnp.float32)
        m = jnp.max(logits, axis=-1, keepdims=True)
        ex = jnp.exp(logits - m)
        p = ex / jnp.sum(ex, axis=-1, keepdims=True)
        ids = jax.lax.broadcasted_iota(jnp.int32, p.shape, 1)
        m1 = jnp.max(p, axis=-1, keepdims=True)
        i1 = jnp.min(jnp.where(p == m1, ids, n_e), axis=-1, keepdims=True)
        p2 = jnp.where(ids == i1, -jnp.inf, p)
        m2 = jnp.max(p2, axis=-1, keepdims=True)
        i2 = jnp.min(jnp.where(p2 == m2, ids, n_e), axis=-1, keepdims=True)
        keep = (ids == i1) | (ids == i2)
        w_scr[...] = jnp.where(keep, p, 0.0)

    ids = jax.lax.broadcasted_iota(jnp.int32, w_scr.shape, 1)
    w_col = jnp.sum(jnp.where(ids == e, w_scr[...], 0.0), axis=-1,
                    keepdims=True)
    h = jax.lax.dot_general(
        x, w1_ref[0], (((1,), (1,)), ((), ())),
        preferred_element_type=jnp.float32)
    h = h * jax.nn.sigmoid(h) * w_col
    y = jax.lax.dot_general(
        h, w2_ref[0], (((1,), (1,)), ((), ())),
        preferred_element_type=jnp.float32)

    @pl.when(e == 0)
    def _():
        out_ref[...] = y

    @pl.when(e != 0)
    def _():
        out_ref[...] += y


@jax.jit
def kernel(x, Wg, W1, W2):
    b, t, h = x.shape
    nt = b * t
    e, f, _ = W1.shape
    x_flat = x.reshape(nt, h)

    out = pl.pallas_call(
        _dense_body,
        grid=(nt // _M, e),
        in_specs=[
            pl.BlockSpec((_M, h), lambda t_, e_: (t_, 0)),
            pl.BlockSpec((e, h), lambda t_, e_: (0, 0)),
            pl.BlockSpec((1, f, h), lambda t_, e_: (e_, 0, 0)),
            pl.BlockSpec((1, h, f), lambda t_, e_: (e_, 0, 0)),
        ],
        out_specs=pl.BlockSpec((_M, h), lambda t_, e_: (t_, 0)),
        out_shape=jax.ShapeDtypeStruct((nt, h), jnp.float32),
        scratch_shapes=[pltpu.VMEM((_M, e), jnp.float32)],
    )(x_flat, Wg, W1, W2)
    return out.reshape(b, t, h)


# speedup vs baseline: 1.2835x; 1.2835x over previous
import functools
import jax
import jax.numpy as jnp
from jax.experimental import pallas as pl
from jax.experimental.pallas import tpu as pltpu

_M = 256


def _dense_body(x_ref, wg_ref, w1_ref, w2_ref, out_ref, w_scr):
    e = pl.program_id(1)
    n_e = pl.num_programs(1)
    x = x_ref[...]

    @pl.when(e == 0)
    def _():
        logits = jax.lax.dot_general(
            x, wg_ref[...], (((1,), (1,)), ((), ())),
            preferred_element_type=jnp.float32)
        m = jnp.max(logits, axis=-1, keepdims=True)
        ex = jnp.exp(logits - m)
        p = ex / jnp.sum(ex, axis=-1, keepdims=True)
        ids = jax.lax.broadcasted_iota(jnp.int32, p.shape, 1)
        m1 = jnp.max(p, axis=-1, keepdims=True)
        i1 = jnp.min(jnp.where(p == m1, ids, n_e), axis=-1, keepdims=True)
        p2 = jnp.where(ids == i1, -jnp.inf, p)
        m2 = jnp.max(p2, axis=-1, keepdims=True)
        i2 = jnp.min(jnp.where(p2 == m2, ids, n_e), axis=-1, keepdims=True)
        keep = (ids == i1) | (ids == i2)
        w_scr[...] = jnp.where(keep, p, 0.0)

    ids = jax.lax.broadcasted_iota(jnp.int32, w_scr.shape, 1)
    w_col = jnp.sum(jnp.where(ids == e, w_scr[...], 0.0), axis=-1,
                    keepdims=True)
    h = jax.lax.dot_general(
        x.astype(jnp.bfloat16), w1_ref[0], (((1,), (1,)), ((), ())),
        preferred_element_type=jnp.float32)
    h = h * jax.nn.sigmoid(h) * w_col
    y = jax.lax.dot_general(
        h.astype(jnp.bfloat16), w2_ref[0], (((1,), (1,)), ((), ())),
        preferred_element_type=jnp.float32)

    @pl.when(e == 0)
    def _():
        out_ref[...] = y

    @pl.when(e != 0)
    def _():
        out_ref[...] += y


@jax.jit
def kernel(x, Wg, W1, W2):
    b, t, h = x.shape
    nt = b * t
    e, f, _ = W1.shape
    x_flat = x.reshape(nt, h)

    out = pl.pallas_call(
        _dense_body,
        grid=(nt // _M, e),
        in_specs=[
            pl.BlockSpec((_M, h), lambda t_, e_: (t_, 0)),
            pl.BlockSpec((e, h), lambda t_, e_: (0, 0)),
            pl.BlockSpec((1, f, h), lambda t_, e_: (e_, 0, 0)),
            pl.BlockSpec((1, h, f), lambda t_, e_: (e_, 0, 0)),
        ],
        out_specs=pl.BlockSpec((_M, h), lambda t_, e_: (t_, 0)),
        out_shape=jax.ShapeDtypeStruct((nt, h), jnp.float32),
        scratch_shapes=[pltpu.VMEM((_M, e), jnp.float32)],
    )(x_flat, Wg, W1.astype(jnp.bfloat16), W2.astype(jnp.bfloat16))
    return out.reshape(b, t, h)
